# baseline (speedup 1.0000x reference)
import functools

import jax
import jax.numpy as jnp
from jax import lax
from jax.experimental import pallas as pl
from jax.experimental.pallas import tpu as pltpu
from jax.experimental.pallas import tpu_sc as plsc

_NC = 2
_NS = 16
_NW = _NC * _NS

_B, _L, _D = 4, 4096, 768
_N = _B * _L
_PER_W = _N // _NW
_CHUNK = 32
_NCHUNK = _PER_W // _CHUNK
_NBUF = 4
_NGROUP = _NCHUNK // _NBUF


_WPB = _L // _PER_W


@functools.partial(
    pl.kernel,
    mesh=plsc.VectorSubcoreMesh(core_axis_name="c", subcore_axis_name="s"),
    out_type=jax.ShapeDtypeStruct((_B, _L, _D), jnp.float32),
    scratch_types=(
        [pltpu.VMEM((_PER_W,), jnp.int32)]
        + [pltpu.VMEM((_CHUNK, _D), jnp.float32) for _ in range(_NBUF)]
        + [pltpu.SemaphoreType.DMA for _ in range(2 * _NBUF)]
    ),
)
def _sc_gather(table_hbm, idx_hbm, out_hbm, idx_v, *refs):
    rows = refs[:_NBUF]
    gsems = refs[_NBUF:2 * _NBUF]
    ssems = refs[2 * _NBUF:]
    wid = lax.axis_index("s") * _NC + lax.axis_index("c")
    bb = wid // _WPB
    loff = (wid % _WPB) * _PER_W
    pltpu.sync_copy(idx_hbm.at[bb, pl.ds(loff, _PER_W)], idx_v)

    def gstart(c, b):
        pltpu.async_copy(
            table_hbm.at[idx_v.at[pl.ds(c * _CHUNK, _CHUNK)]], rows[b], gsems[b])

    def gwait(c, b):
        pltpu.make_async_copy(
            table_hbm.at[idx_v.at[pl.ds(c * _CHUNK, _CHUNK)]], rows[b],
            gsems[b]).wait()

    def sstart(c, b):
        pltpu.async_copy(
            rows[b], out_hbm.at[bb, pl.ds(loff + c * _CHUNK, _CHUNK)], ssems[b])

    def swait(c, b):
        pltpu.make_async_copy(
            rows[b], out_hbm.at[bb, pl.ds(loff + c * _CHUNK, _CHUNK)],
            ssems[b]).wait()

    gstart(0, 0)
    gstart(1, 1)
    gstart(2, 2)
    gwait(0, 0)
    sstart(0, 0)
    gstart(3, 3)
    for b in range(1, _NBUF):
        gwait(b, b)
        sstart(b, b)
        swait(b - 1, b - 1)
        gstart(b + _NBUF - 1, b - 1)

    @pl.loop(1, _NGROUP - 1)
    def _group(g):
        c0 = g * _NBUF
        gwait(c0, 0)
        sstart(c0, 0)
        swait(c0 - 1, _NBUF - 1)
        gstart(c0 + _NBUF - 1, _NBUF - 1)
        for b in range(1, _NBUF):
            c = c0 + b
            gwait(c, b)
            sstart(c, b)
            swait(c - 1, b - 1)
            gstart(c + _NBUF - 1, b - 1)

    c0 = (_NGROUP - 1) * _NBUF
    gwait(c0, 0)
    sstart(c0, 0)
    swait(c0 - 1, _NBUF - 1)
    gstart(_NCHUNK - 1, _NBUF - 1)
    for b in range(1, _NBUF):
        gwait(c0 + b, b)
        sstart(c0 + b, b)
    for b in range(_NBUF):
        swait(c0 + b, b)


def kernel(src, embedding_table):
    if src.dtype != jnp.int32:
        src = src.astype(jnp.int32)
    return _sc_gather(embedding_table, src)


# speedup vs baseline: 1.0055x; 1.0055x over previous
import functools

import jax
import jax.numpy as jnp
from jax import lax
from jax.experimental import pallas as pl
from jax.experimental.pallas import tpu as pltpu
from jax.experimental.pallas import tpu_sc as plsc

_NC = 2
_NS = 16
_NW = _NC * _NS

_B, _L, _D = 4, 4096, 768
_N = _B * _L
_PER_W = _N // _NW
_CHUNK = 16
_NCHUNK = _PER_W // _CHUNK
_NBUF = 4
_NGROUP = _NCHUNK // _NBUF


_WPB = _L // _PER_W


@functools.partial(
    pl.kernel,
    mesh=plsc.VectorSubcoreMesh(core_axis_name="c", subcore_axis_name="s"),
    out_type=jax.ShapeDtypeStruct((_B, _L, _D), jnp.float32),
    scratch_types=(
        [pltpu.VMEM((_PER_W,), jnp.int32)]
        + [pltpu.VMEM((_CHUNK, _D), jnp.float32) for _ in range(_NBUF)]
        + [pltpu.SemaphoreType.DMA for _ in range(2 * _NBUF)]
    ),
)
def _sc_gather(table_hbm, idx_hbm, out_hbm, idx_v, *refs):
    rows = refs[:_NBUF]
    gsems = refs[_NBUF:2 * _NBUF]
    ssems = refs[2 * _NBUF:]
    wid = lax.axis_index("s") * _NC + lax.axis_index("c")
    bb = wid // _WPB
    loff = (wid % _WPB) * _PER_W
    pltpu.sync_copy(idx_hbm.at[bb, pl.ds(loff, _PER_W)], idx_v)

    def gstart(c, b):
        pltpu.async_copy(
            table_hbm.at[idx_v.at[pl.ds(c * _CHUNK, _CHUNK)]], rows[b], gsems[b])

    def gwait(c, b):
        pltpu.make_async_copy(
            table_hbm.at[idx_v.at[pl.ds(c * _CHUNK, _CHUNK)]], rows[b],
            gsems[b]).wait()

    def sstart(c, b):
        pltpu.async_copy(
            rows[b], out_hbm.at[bb, pl.ds(loff + c * _CHUNK, _CHUNK)], ssems[b])

    def swait(c, b):
        pltpu.make_async_copy(
            rows[b], out_hbm.at[bb, pl.ds(loff + c * _CHUNK, _CHUNK)],
            ssems[b]).wait()

    gstart(0, 0)
    gstart(1, 1)
    gstart(2, 2)
    gwait(0, 0)
    sstart(0, 0)
    gstart(3, 3)
    for b in range(1, _NBUF):
        gwait(b, b)
        sstart(b, b)
        swait(b - 1, b - 1)
        gstart(b + _NBUF - 1, b - 1)

    @pl.loop(1, _NGROUP - 1)
    def _group(g):
        c0 = g * _NBUF
        gwait(c0, 0)
        sstart(c0, 0)
        swait(c0 - 1, _NBUF - 1)
        gstart(c0 + _NBUF - 1, _NBUF - 1)
        for b in range(1, _NBUF):
            c = c0 + b
            gwait(c, b)
            sstart(c, b)
            swait(c - 1, b - 1)
            gstart(c + _NBUF - 1, b - 1)

    c0 = (_NGROUP - 1) * _NBUF
    gwait(c0, 0)
    sstart(c0, 0)
    swait(c0 - 1, _NBUF - 1)
    gstart(_NCHUNK - 1, _NBUF - 1)
    for b in range(1, _NBUF):
        gwait(c0 + b, b)
        sstart(c0 + b, b)
    for b in range(_NBUF):
        swait(c0 + b, b)


def kernel(src, embedding_table):
    if src.dtype != jnp.int32:
        src = src.astype(jnp.int32)
    return _sc_gather(embedding_table, src)
